# baseline (speedup 1.0000x reference)
import functools

import jax
import jax.numpy as jnp
from jax import lax
from jax.experimental import pallas as pl
from jax.experimental.pallas import tpu as pltpu, tpu_sc as plsc

VOCAB = 100000
MAX_LEN = 200
DIM = 32
BATCH = 4096

NC, NS, L = 2, 16, 16
NW = NC * NS
ROWS = BATCH * MAX_LEN
SEQ_PER_W = BATCH // NW
SEQ_PER_GROUP = 2
GROUP = SEQ_PER_GROUP * MAX_LEN
NGROUPS = SEQ_PER_W // SEQ_PER_GROUP
GCHUNK = 80
ROWS128 = ROWS * DIM // 128
GROW = GROUP * DIM // 128
QPS = MAX_LEN // 4
NBUF = 4
PRIME = NBUF - 1


def _make_kernel():
  mesh = plsc.VectorSubcoreMesh(
      core_axis_name="c", subcore_axis_name="s", num_cores=NC, num_subcores=NS
  )

  @functools.partial(
      pl.kernel,
      mesh=mesh,
      compiler_params=pltpu.CompilerParams(use_tc_tiling_on_sc=False),
      out_type=jax.ShapeDtypeStruct((ROWS128, 128), jnp.float32),
      scratch_types=[
          [pltpu.VMEM((SEQ_PER_GROUP, MAX_LEN), jnp.int32) for _ in range(NBUF)],
          [pltpu.VMEM((GROUP, DIM), jnp.float32) for _ in range(NBUF)],
          [pltpu.VMEM((GROW, 128), jnp.float32) for _ in range(NBUF)],
          pltpu.VMEM((MAX_LEN, DIM), jnp.float32),
          [pltpu.SemaphoreType.DMA for _ in range(NBUF)],
          [pltpu.SemaphoreType.DMA for _ in range(NBUF)],
      ],
  )
  def k(x_hbm, table_hbm, pos_hbm, out_hbm, idx_v, rows_v, w_v, pos_v,
        gsem, wsem):
    wid = lax.axis_index("s") * NC + lax.axis_index("c")
    pltpu.sync_copy(pos_hbm, pos_v)
    w_seq = wid * SEQ_PER_W

    def fire_group(g, b):
      seq0 = w_seq + g * SEQ_PER_GROUP
      pltpu.sync_copy(x_hbm.at[pl.ds(seq0, SEQ_PER_GROUP)], idx_v[b])
      for s in range(SEQ_PER_GROUP):
        off = 0
        for c in (GCHUNK, GCHUNK, MAX_LEN - 2 * GCHUNK):
          pltpu.async_copy(
              table_hbm.at[idx_v[b].at[s].at[pl.ds(off, c)]],
              rows_v[b].at[pl.ds(s * MAX_LEN + off, c)],
              gsem[b],
          )
          off += c

    def wait_gathers(b):
      pltpu.make_async_copy(
          out_hbm.at[pl.ds(0, GROW)], w_v[b], gsem[b]
      ).wait()

    def wait_write(b):
      pltpu.make_async_copy(
          w_v[b], out_hbm.at[pl.ds(0, GROW)], wsem[b]
      ).wait()

    for p in range(PRIME):
      fire_group(p, p)

    def h_body(h, carry):
      for b in range(NBUF):
        g = h * NBUF + b
        gf = g + PRIME
        bf = (b + PRIME) % NBUF

        @pl.when(gf < NGROUPS)
        def _():
          fire_group(gf, bf)

        wait_gathers(b)

        @pl.when(g >= NBUF)
        def _():
          wait_write(b)

        def add_body(q, c):
          for kq in range(4):
            l = 4 * q + kq
            p0 = pos_v[l, pl.ds(0, L)]
            p1 = pos_v[l, pl.ds(L, L)]
            for s in range(SEQ_PER_GROUP):
              t = s * MAX_LEN + l
              r = s * QPS + q
              w_v[b][r, pl.ds(32 * kq, L)] = rows_v[b][t, pl.ds(0, L)] + p0
              w_v[b][r, pl.ds(32 * kq + L, L)] = (
                  rows_v[b][t, pl.ds(L, L)] + p1)
          return c
        lax.fori_loop(0, QPS, add_body, 0)

        row0 = (w_seq + g * SEQ_PER_GROUP) * QPS
        pltpu.async_copy(w_v[b], out_hbm.at[pl.ds(row0, GROW)], wsem[b])
      return carry

    lax.fori_loop(0, NGROUPS // NBUF, h_body, 0)

    for b in range(NBUF):
      wait_write(b)

  return k


BB = 16


def _make_tc_relayout():
  def body(in_ref, out_ref):
    d = in_ref[...]
    parts = [
        d[:, 32 * k:32 * (k + 1)].reshape(BB * QPS, 1, DIM) for k in range(4)
    ]
    out_ref[...] = jnp.concatenate(parts, axis=1).reshape(BB, MAX_LEN, DIM)

  return pl.pallas_call(
      body,
      grid=(BATCH // BB,),
      in_specs=[pl.BlockSpec((BB * QPS, 128), lambda i: (i, 0))],
      out_specs=pl.BlockSpec((BB, MAX_LEN, DIM), lambda i: (i, 0, 0)),
      out_shape=jax.ShapeDtypeStruct((BATCH, MAX_LEN, DIM), jnp.float32),
  )


S2 = 4
G2 = SEQ_PER_W // S2
GR2 = S2 * QPS
SST = 2
NSUB = S2 // SST


def _make_relayout_kernel():
  mesh = plsc.VectorSubcoreMesh(
      core_axis_name="c", subcore_axis_name="s", num_cores=NC, num_subcores=NS
  )

  @functools.partial(
      pl.kernel,
      mesh=mesh,
      compiler_params=pltpu.CompilerParams(use_tc_tiling_on_sc=True),
      out_type=jax.ShapeDtypeStruct((BATCH, MAX_LEN, DIM), jnp.float32),
      scratch_types=[
          pltpu.VMEM((GR2, 128), jnp.float32),
          [pltpu.VMEM((SST, MAX_LEN, DIM), jnp.float32) for _ in range(NSUB)],
          pltpu.SemaphoreType.DMA,
          [pltpu.SemaphoreType.DMA for _ in range(NSUB)],
      ],
  )
  def k(mid_hbm, out_hbm, in_v, st_v, rsem, wsem):
    wid = lax.axis_index("s") * NC + lax.axis_index("c")
    w_seq = wid * SEQ_PER_W

    def fire_read(g):
      row0 = (w_seq + g * S2) * QPS
      pltpu.async_copy(mid_hbm.at[pl.ds(row0, GR2)], in_v, rsem)

    def wait_read():
      pltpu.make_async_copy(
          mid_hbm.at[pl.ds(0, GR2)], in_v, rsem
      ).wait()

    def wait_write(b):
      pltpu.make_async_copy(
          st_v[b], out_hbm.at[pl.ds(0, SST)], wsem[b]
      ).wait()

    fire_read(0)

    def h_body(g, carry):
      wait_read()

      for b in range(NSUB):
        @pl.when(g >= 1)
        def _():
          wait_write(b)

        def cp_body(q, c):
          for kq in range(4):
            l = 4 * q + kq
            for s in range(SST):
              r = (b * SST + s) * QPS + q
              st_v[b][s, l, pl.ds(0, L)] = in_v[r, pl.ds(32 * kq, L)]
              st_v[b][s, l, pl.ds(L, L)] = in_v[r, pl.ds(32 * kq + L, L)]
          return c
        lax.fori_loop(0, QPS, cp_body, 0)

        @pl.when(jnp.logical_and(g + 1 < G2, b == NSUB - 1))
        def _():
          fire_read(g + 1)

        pltpu.async_copy(
            st_v[b],
            out_hbm.at[pl.ds(w_seq + g * S2 + b * SST, SST)],
            wsem[b],
        )
      return carry

    lax.fori_loop(0, G2, h_body, 0)

    for b in range(NSUB):
      wait_write(b)

  return k


_kernel_cache = []


def kernel(x, token_table, pos_table):
  if not _kernel_cache:
    _kernel_cache.append((_make_kernel(), _make_tc_relayout()))
  k1, k2 = _kernel_cache[0]
  mid = k1(x.astype(jnp.int32), token_table, pos_table)
  return k2(mid)


# speedup vs baseline: 2.0304x; 2.0304x over previous
import functools

import jax
import jax.numpy as jnp
from jax import lax
from jax.experimental import pallas as pl
from jax.experimental.pallas import tpu as pltpu, tpu_sc as plsc

VOCAB = 100000
MAX_LEN = 200
DIM = 32
BATCH = 4096

NC, NS, L = 2, 16, 16
NW = NC * NS
NCALLS = 8
BC = BATCH // NCALLS
SEQ_PER_W = BC // NW
SEQ_PER_GROUP = 4
GROUP = SEQ_PER_GROUP * MAX_LEN
NGROUPS = SEQ_PER_W // SEQ_PER_GROUP
GCHUNK = 80
NBUF = 4
PRIME = NBUF - 1


def _make_kernel():
  mesh = plsc.VectorSubcoreMesh(
      core_axis_name="c", subcore_axis_name="s", num_cores=NC, num_subcores=NS
  )

  @functools.partial(
      pl.kernel,
      mesh=mesh,
      compiler_params=pltpu.CompilerParams(use_tc_tiling_on_sc=False),
      out_type=jax.ShapeDtypeStruct((BC, MAX_LEN, DIM), jnp.float32),
      scratch_types=[
          [pltpu.VMEM((SEQ_PER_GROUP, MAX_LEN), jnp.int32) for _ in range(NBUF)],
          [pltpu.VMEM((SEQ_PER_GROUP, MAX_LEN, DIM), jnp.float32)
           for _ in range(NBUF)],
          pltpu.VMEM((MAX_LEN, DIM), jnp.float32),
          [pltpu.SemaphoreType.DMA for _ in range(NBUF)],
          [pltpu.SemaphoreType.DMA for _ in range(NBUF)],
      ],
  )
  def k(x_hbm, table_hbm, pos_hbm, out_hbm, idx_v, rows_v, pos_v, gsem, wsem):
    wid = lax.axis_index("s") * NC + lax.axis_index("c")
    pltpu.sync_copy(pos_hbm, pos_v)
    w_seq = wid * SEQ_PER_W

    def fire_group(g, b):
      seq0 = w_seq + g * SEQ_PER_GROUP
      pltpu.sync_copy(x_hbm.at[pl.ds(seq0, SEQ_PER_GROUP)], idx_v[b])
      for s in range(SEQ_PER_GROUP):
        off = 0
        for c in (GCHUNK, GCHUNK, MAX_LEN - 2 * GCHUNK):
          pltpu.async_copy(
              table_hbm.at[idx_v[b].at[s].at[pl.ds(off, c)]],
              rows_v[b].at[s].at[pl.ds(off, c)],
              gsem[b],
          )
          off += c

    def wait_gathers(b):
      pltpu.make_async_copy(
          out_hbm.at[pl.ds(0, SEQ_PER_GROUP)], rows_v[b], gsem[b]
      ).wait()

    def wait_write(b):
      pltpu.make_async_copy(
          rows_v[b], out_hbm.at[pl.ds(0, SEQ_PER_GROUP)], wsem[b]
      ).wait()

    for p in range(PRIME):
      fire_group(p, p)

    def h_body(h, carry):
      for b in range(NBUF):
        g = h * NBUF + b
        bf = (b + PRIME) % NBUF
        gf = g + PRIME

        @pl.when(jnp.logical_and(gf < NGROUPS, g >= 1))
        def _():
          wait_write(bf)
          fire_group(gf, bf)

        @pl.when(jnp.logical_and(gf < NGROUPS, g < 1))
        def _():
          fire_group(gf, bf)

        wait_gathers(b)

        def add_body(l, c):
          p0 = pos_v[l, pl.ds(0, L)]
          p1 = pos_v[l, pl.ds(L, L)]
          for s in range(SEQ_PER_GROUP):
            rows_v[b][s, l, pl.ds(0, L)] = rows_v[b][s, l, pl.ds(0, L)] + p0
            rows_v[b][s, l, pl.ds(L, L)] = rows_v[b][s, l, pl.ds(L, L)] + p1
          return c
        lax.fori_loop(0, MAX_LEN, add_body, 0)

        pltpu.async_copy(
            rows_v[b],
            out_hbm.at[pl.ds(w_seq + g * SEQ_PER_GROUP, SEQ_PER_GROUP)],
            wsem[b],
        )
      return carry

    lax.fori_loop(0, NGROUPS // NBUF, h_body, 0)

    for b in range(NBUF):
      wait_write(b)

  return k


_kernel_cache = []


def kernel(x, token_table, pos_table):
  if not _kernel_cache:
    _kernel_cache.append(_make_kernel())
  k = _kernel_cache[0]
  xi = x.astype(jnp.int32)
  parts = [
      k(xi[i * BC:(i + 1) * BC], token_table, pos_table)
      for i in range(NCALLS)
  ]
  return jnp.concatenate(parts, axis=0)
